# Initial kernel scaffold; baseline (speedup 1.0000x reference)
import jax
import jax.numpy as jnp
from jax.experimental import pallas as pl


def kernel(similarity, text_en, text_input):
    raise NotImplementedError("write your pallas kernel here")


# speedup vs baseline: 1.6304x; 1.6304x over previous
import functools

import jax
import jax.numpy as jnp
from jax import lax
from jax.experimental import pallas as pl
from jax.experimental.pallas import tpu as pltpu
from jax.experimental.pallas import tpu_sc as plsc

B = 128
N = 32768
K = 16


_NC, _NS = 2, 16
_NW = _NC * _NS
_PER_W = (B * K) // _NW
_ROWS_W = B // _NW


def _sc_gather_body(te_hbm, sim_hbm, out_hbm, idx_v, val_v, sem):
    wid = lax.axis_index("s") * _NC + lax.axis_index("c")
    base = wid * _PER_W
    pltpu.sync_copy(te_hbm.at[pl.ds(base, _PER_W)], idx_v)
    for r in range(_ROWS_W):
        row = wid * _ROWS_W + r
        chunk = idx_v[pl.ds(r * K, 16)]
        idx_v[pl.ds(r * K, 16)] = chunk + row * N
    pltpu.async_copy(sim_hbm.at[idx_v], val_v, sem).wait()
    pltpu.sync_copy(val_v, out_hbm.at[pl.ds(base, _PER_W)])


def _sc_gather(te_flat, sim_flat):
    mesh = plsc.VectorSubcoreMesh(core_axis_name="c", subcore_axis_name="s")
    kfn = pl.kernel(
        _sc_gather_body,
        mesh=mesh,
        out_type=jax.ShapeDtypeStruct((B * K,), jnp.float32),
        scratch_types=[
            pltpu.VMEM((_PER_W,), jnp.int32),
            pltpu.VMEM((_PER_W,), jnp.float32),
            pltpu.SemaphoreType.DMA,
        ],
    )
    return kfn(te_flat, sim_flat)


_RB = 8
_STEPS = B // _RB


def _tc_body(sim_ref, xv_ref, te_ref, out_ref):
    b = pl.program_id(0)
    x = sim_ref[...]
    y = 100.0 * x
    m = jnp.max(y, axis=1, keepdims=True)
    e = jnp.exp(y - m)
    z = jnp.sum(e, axis=1, keepdims=True)
    s = e / z

    yv = 100.0 * xv_ref[...]
    ev = jnp.exp(yv - m)
    sv = ev / z
    te = te_ref[...]

    col = lax.broadcasted_iota(jnp.int32, (_RB, N), 1)

    inter = jnp.zeros((_RB, 1), jnp.float32)
    for i in range(K):
        svi = sv[:, i:i + 1]
        ti = te[:, i:i + 1]
        gt = jnp.sum(jnp.where(s > svi, 1.0, 0.0), axis=1, keepdims=True)
        eq = jnp.sum(
            jnp.where((s == svi) & (col < ti), 1.0, 0.0), axis=1, keepdims=True)
        present = (gt + eq) < float(K)
        dup = jnp.zeros((_RB, 1), jnp.bool_)
        for j in range(i):
            dup = dup | (te[:, j:j + 1] == ti)
        inter = inter + jnp.where(present & (~dup), 1.0, 0.0)

    acc = inter / float(K) * 100.0
    acc = jnp.round(acc * 1e6) / 1e6
    total = jnp.sum(acc, axis=0, keepdims=True)

    @pl.when(b == 0)
    def _init():
        out_ref[...] = jnp.zeros((1, 1), jnp.float32)

    out_ref[...] += total

    @pl.when(b == _STEPS - 1)
    def _fin():
        out_ref[...] = out_ref[...] / float(B)


def _tc_stats(similarity, x_gathered, te):
    return pl.pallas_call(
        _tc_body,
        grid=(_STEPS,),
        in_specs=[
            pl.BlockSpec((_RB, N), lambda b: (b, 0)),
            pl.BlockSpec((_RB, K), lambda b: (b, 0)),
            pl.BlockSpec((_RB, K), lambda b: (b, 0)),
        ],
        out_specs=pl.BlockSpec((1, 1), lambda b: (0, 0)),
        out_shape=jax.ShapeDtypeStruct((1, 1), jnp.float32),
    )(similarity, x_gathered, te)


def kernel(similarity, text_en, text_input):
    del text_input
    te = text_en.astype(jnp.int32)
    gathered = _sc_gather(te.reshape(-1), similarity.reshape(-1))
    out = _tc_stats(similarity, gathered.reshape(B, K), te)
    return out.reshape(())


# speedup vs baseline: 3.3121x; 2.0315x over previous
import functools

import jax
import jax.numpy as jnp
from jax import lax
from jax.experimental import pallas as pl
from jax.experimental.pallas import tpu as pltpu
from jax.experimental.pallas import tpu_sc as plsc

B = 128
N = 32768
K = 16


_NC, _NS = 2, 16
_NW = _NC * _NS
_PER_W = (B * K) // _NW
_ROWS_W = B // _NW


def _sc_gather_body(te_hbm, sim_hbm, out_hbm, idx_v, val_v, sem):
    wid = lax.axis_index("s") * _NC + lax.axis_index("c")
    base = wid * _PER_W
    pltpu.sync_copy(te_hbm.at[pl.ds(base, _PER_W)], idx_v)
    for r in range(_ROWS_W):
        row = wid * _ROWS_W + r
        chunk = idx_v[pl.ds(r * K, 16)]
        idx_v[pl.ds(r * K, 16)] = chunk + row * N
    pltpu.async_copy(sim_hbm.at[idx_v], val_v, sem).wait()
    pltpu.sync_copy(val_v, out_hbm.at[pl.ds(base, _PER_W)])


def _sc_gather(te_flat, sim_flat):
    mesh = plsc.VectorSubcoreMesh(core_axis_name="c", subcore_axis_name="s")
    kfn = pl.kernel(
        _sc_gather_body,
        mesh=mesh,
        out_type=jax.ShapeDtypeStruct((B * K,), jnp.float32),
        scratch_types=[
            pltpu.VMEM((_PER_W,), jnp.int32),
            pltpu.VMEM((_PER_W,), jnp.float32),
            pltpu.SemaphoreType.DMA,
        ],
    )
    return kfn(te_flat, sim_flat)


_RB = 8
_STEPS = B // _RB


def _tc_body(sim_ref, xv_ref, te_ref, out_ref, inter_ref):
    b = pl.program_id(0)
    x = sim_ref[...]
    y = 100.0 * x
    m = jnp.max(y, axis=1, keepdims=True)
    e = jnp.exp(y - m)
    z = jnp.sum(e, axis=1, keepdims=True)
    s = e / z

    yv = 100.0 * xv_ref[...]
    ev = jnp.exp(yv - m)
    sv = ev / z
    te = te_ref[...]

    colk = lax.broadcasted_iota(jnp.int32, (_RB, K), 1)
    dup = jnp.zeros((_RB, K), jnp.bool_)
    for j in range(K - 1):
        dup = dup | ((te == te[:, j:j + 1]) & (colk > j))

    nz = jnp.sum(jnp.where(s > 0.0, 1.0, 0.0), axis=1, keepdims=True)
    s16 = s[:, :K] > 0.0
    nzb = jnp.zeros((_RB, K), jnp.float32)
    for j in range(K):
        nzb = nzb + jnp.where(s16[:, j:j + 1] & (j < te), 1.0, 0.0)
    tef = te.astype(jnp.float32)
    present0 = (te < K) & ((nz + tef - nzb) < float(K))
    inter_ref[...] = jnp.sum(
        jnp.where(present0 & (~dup), 1.0, 0.0), axis=1, keepdims=True)

    any_pos = jnp.any(sv > 0.0)

    @pl.when(any_pos)
    def _full():
        col = lax.broadcasted_iota(jnp.int32, (_RB, N), 1)
        inter = jnp.zeros((_RB, 1), jnp.float32)
        for i in range(K):
            svi = sv[:, i:i + 1]
            ti = te[:, i:i + 1]
            gt = jnp.sum(jnp.where(s > svi, 1.0, 0.0), axis=1, keepdims=True)
            eq = jnp.sum(
                jnp.where((s == svi) & (col < ti), 1.0, 0.0),
                axis=1, keepdims=True)
            present = (gt + eq) < float(K)
            inter = inter + jnp.where(
                present & (~dup[:, i:i + 1]), 1.0, 0.0)
        inter_ref[...] = inter

    acc = inter_ref[...] / float(K) * 100.0
    acc = jnp.round(acc * 1e6) / 1e6
    total = jnp.sum(acc, axis=0, keepdims=True)

    @pl.when(b == 0)
    def _init():
        out_ref[...] = jnp.zeros((1, 1), jnp.float32)

    out_ref[...] += total

    @pl.when(b == _STEPS - 1)
    def _fin():
        out_ref[...] = out_ref[...] / float(B)


def _tc_stats(similarity, x_gathered, te):
    return pl.pallas_call(
        _tc_body,
        grid=(_STEPS,),
        in_specs=[
            pl.BlockSpec((_RB, N), lambda b: (b, 0)),
            pl.BlockSpec((_RB, K), lambda b: (b, 0)),
            pl.BlockSpec((_RB, K), lambda b: (b, 0)),
        ],
        out_specs=pl.BlockSpec((1, 1), lambda b: (0, 0)),
        out_shape=jax.ShapeDtypeStruct((1, 1), jnp.float32),
        scratch_shapes=[pltpu.VMEM((_RB, 1), jnp.float32)],
    )(similarity, x_gathered, te)


def kernel(similarity, text_en, text_input):
    del text_input
    te = text_en.astype(jnp.int32)
    gathered = _sc_gather(te.reshape(-1), similarity.reshape(-1))
    out = _tc_stats(similarity, gathered.reshape(B, K), te)
    return out.reshape(())
